# baseline (speedup 1.0000x reference)
import functools

import jax
import jax.numpy as jnp
from jax import lax
from jax.experimental import pallas as pl
from jax.experimental.pallas import tpu as pltpu
from jax.experimental.pallas import tpu_sc as plsc

N_NODES = 10000
N_PAD = 10240
N_EDGES = 320000
D_FEAT = 128
GAM = 0.1
LAM = 1.0
ALP = 0.5

NC = 2
NS = 16
NW = NC * NS
FPW = D_FEAT // NW
SLAB = FPW * N_NODES
PSLAB = FPW * N_PAD
CH = 3200
NCH = N_EDGES // CH


def _pack_body(ei_ref, em_ref, packed_ref, ms_ref, l1_ref, en_ref):
    em = em_ref[...]
    ms = jax.nn.sigmoid(em)
    keep = (em > 0.0).astype(jnp.int32)
    dstp = ei_ref[1, :] + keep * PSLAB
    packed_ref[...] = ei_ref[0, :] + dstp * 16384
    ms_ref[...] = ms
    l1_ref[...] = jnp.sum(ms)[None, None]
    en_ref[...] = jnp.sum(keep)[None, None]


def _pack(edge_index, edge_mask):
    return pl.pallas_call(
        _pack_body,
        out_shape=(
            jax.ShapeDtypeStruct((N_EDGES,), jnp.int32),
            jax.ShapeDtypeStruct((N_EDGES,), jnp.float32),
            jax.ShapeDtypeStruct((1, 1), jnp.float32),
            jax.ShapeDtypeStruct((1, 1), jnp.int32),
        ),
    )(edge_index, edge_mask)


def _sc_body(packed_hbm, xflat_hbm, agg1_hbm, agg2_hbm,
             xv, accv, eb0, eb1, sem0, sem1):
    w = lax.axis_index("s") * NC + lax.axis_index("c")

    pltpu.async_copy(packed_hbm.at[pl.ds(0, CH)], eb0, sem0)

    pltpu.sync_copy(xflat_hbm.at[pl.ds(w * SLAB, SLAB)], xv)

    @plsc.parallel_loop(0, 2 * PSLAB, step=16, unroll=8)
    def _(i):
        accv[pl.ds(i, 16)] = jnp.zeros((16,), jnp.float32)

    def process(eb):
        @plsc.parallel_loop(0, CH, step=16, unroll=16)
        def _(i):
            p = eb[pl.ds(i, 16)]
            srci = lax.bitwise_and(p, 16383)
            dbase = lax.shift_right_logical(p, 14)
            for j in range(FPW):
                v = plsc.load_gather(xv, [srci + (j * N_NODES)])
                plsc.addupdate_scatter(accv, [dbase + (j * N_PAD)], v)

    def pair(cc, _):
        c0 = cc * 2
        pltpu.async_copy(packed_hbm.at[pl.ds((c0 + 1) * CH, CH)], eb1, sem1)
        pltpu.make_async_copy(packed_hbm.at[pl.ds(c0 * CH, CH)], eb0, sem0).wait()
        process(eb0)
        nxt = jnp.minimum(c0 + 2, NCH - 2)
        pltpu.async_copy(packed_hbm.at[pl.ds(nxt * CH, CH)], eb0, sem0)
        pltpu.make_async_copy(packed_hbm.at[pl.ds((c0 + 1) * CH, CH)], eb1, sem1).wait()
        process(eb1)
        return 0

    lax.fori_loop(0, NCH // 2, pair, 0)
    pltpu.make_async_copy(packed_hbm.at[pl.ds(0, CH)], eb0, sem0).wait()

    base = w * PSLAB
    c1 = pltpu.async_copy(accv.at[pl.ds(0, PSLAB)], agg1_hbm.at[pl.ds(base, PSLAB)], sem0)
    c2 = pltpu.async_copy(accv.at[pl.ds(PSLAB, PSLAB)], agg2_hbm.at[pl.ds(base, PSLAB)], sem1)
    c1.wait()
    c2.wait()


def _sc_scatter(packed_flat, xflat):
    mesh = plsc.VectorSubcoreMesh(
        core_axis_name="c", subcore_axis_name="s",
        num_cores=NC, num_subcores=NS)
    f = functools.partial(
        pl.kernel,
        out_type=(
            jax.ShapeDtypeStruct((D_FEAT * N_PAD,), jnp.float32),
            jax.ShapeDtypeStruct((D_FEAT * N_PAD,), jnp.float32),
        ),
        mesh=mesh,
        compiler_params=pltpu.CompilerParams(needs_layout_passes=False),
        scratch_types=[
            pltpu.VMEM((SLAB,), jnp.float32),
            pltpu.VMEM((2 * PSLAB,), jnp.float32),
            pltpu.VMEM((CH,), jnp.int32),
            pltpu.VMEM((CH,), jnp.int32),
            pltpu.SemaphoreType.DMA,
            pltpu.SemaphoreType.DMA,
        ],
    )(_sc_body)
    return f(packed_flat, xflat)


def _head_body(a1_ref, a2_ref, w1t_ref, b1_ref, w2_ref, l1_ref, loss_ref):
    w1t = w1t_ref[...]
    b1 = b1_ref[...]
    w2 = w2_ref[...]
    valid = lax.broadcasted_iota(jnp.int32, (1, N_PAD), 1) < N_NODES
    h1 = jnp.maximum(
        jnp.dot(w1t, a1_ref[...], preferred_element_type=jnp.float32) + b1, 0.0)
    h2 = jnp.maximum(
        jnp.dot(w1t, a2_ref[...], preferred_element_type=jnp.float32) + b1, 0.0)
    s1 = jnp.sum(jnp.where(valid, h1 * w2, 0.0))
    s2 = jnp.sum(jnp.where(valid, h2 * w2, 0.0))
    pred1 = jax.nn.sigmoid(s1 / N_NODES)
    pred2 = jax.nn.sigmoid(s2 / N_NODES)
    bpr1 = jnp.maximum(GAM + 0.5 - pred1, 0.0)
    bpr2 = jnp.maximum(GAM + pred2 - 0.5, 0.0)
    loss_ref[...] = l1_ref[...] + LAM * (ALP * bpr1 + (1.0 - ALP) * bpr2)


def _head(a1, a2, w1t, b1c, w2c, l1):
    return pl.pallas_call(
        _head_body,
        out_shape=jax.ShapeDtypeStruct((1, 1), jnp.float32),
    )(a1, a2, w1t, b1c, w2c, l1)


def kernel(x, edge_mask, W1, b1, w2, edge_index):
    packed, ms, l1, expn = _pack(edge_index, edge_mask)

    xflat = x.T.reshape(-1)
    agg1f, agg2f = _sc_scatter(packed, xflat)

    loss = _head(agg1f.reshape(D_FEAT, N_PAD), agg2f.reshape(D_FEAT, N_PAD),
                 W1.T, b1.reshape(D_FEAT, 1), w2.reshape(D_FEAT, 1), l1)
    return loss[0, 0], ms, expn[0, 0]


# speedup vs baseline: 1.0521x; 1.0521x over previous
import functools

import jax
import jax.numpy as jnp
from jax import lax
from jax.experimental import pallas as pl
from jax.experimental.pallas import tpu as pltpu
from jax.experimental.pallas import tpu_sc as plsc

N_NODES = 10000
N_PAD = 10240
N_EDGES = 320000
D_FEAT = 128
GAM = 0.1
LAM = 1.0
ALP = 0.5

NC = 2
NS = 16
NW = NC * NS
FPW = D_FEAT // NW
SLAB = FPW * N_NODES
PSLAB = FPW * N_PAD
CH = 3200
NCH = N_EDGES // CH


def _pack_body(ei_ref, em_ref, packed_ref, ms_ref, l1_ref, en_ref):
    em = em_ref[...]
    ms = jax.nn.sigmoid(em)
    keep = (em > 0.0).astype(jnp.int32)
    dstp = ei_ref[1, :] + keep * PSLAB
    packed_ref[...] = ei_ref[0, :] + dstp * 16384
    ms_ref[...] = ms
    l1_ref[...] = jnp.sum(ms)[None, None]
    en_ref[...] = jnp.sum(keep)[None, None]


def _pack(edge_index, edge_mask):
    return pl.pallas_call(
        _pack_body,
        out_shape=(
            jax.ShapeDtypeStruct((N_EDGES,), jnp.int32),
            jax.ShapeDtypeStruct((N_EDGES,), jnp.float32),
            jax.ShapeDtypeStruct((1, 1), jnp.float32),
            jax.ShapeDtypeStruct((1, 1), jnp.int32),
        ),
    )(edge_index, edge_mask)


def _sc_body(packed_hbm, xflat_hbm, agg1_hbm, agg2_hbm,
             xv, accv, eb0, eb1, sem0, sem1):
    w = lax.axis_index("s") * NC + lax.axis_index("c")

    pltpu.async_copy(packed_hbm.at[pl.ds(0, CH)], eb0, sem0)

    pltpu.sync_copy(xflat_hbm.at[pl.ds(w * SLAB, SLAB)], xv)

    @plsc.parallel_loop(0, 2 * PSLAB, step=16, unroll=8)
    def _(i):
        accv[pl.ds(i, 16)] = jnp.zeros((16,), jnp.float32)

    def process(eb):
        @plsc.parallel_loop(0, CH, step=16, unroll=8)
        def _(i):
            p = eb[pl.ds(i, 16)]
            srci = lax.bitwise_and(p, 16383)
            dbase = lax.shift_right_logical(p, 14)
            for j in range(FPW):
                v = plsc.load_gather(xv, [srci + (j * N_NODES)])
                plsc.addupdate_scatter(accv, [dbase + (j * N_PAD)], v)

    def pair(cc, _):
        c0 = cc * 2
        pltpu.async_copy(packed_hbm.at[pl.ds((c0 + 1) * CH, CH)], eb1, sem1)
        pltpu.make_async_copy(packed_hbm.at[pl.ds(c0 * CH, CH)], eb0, sem0).wait()
        process(eb0)
        nxt = jnp.minimum(c0 + 2, NCH - 2)
        pltpu.async_copy(packed_hbm.at[pl.ds(nxt * CH, CH)], eb0, sem0)
        pltpu.make_async_copy(packed_hbm.at[pl.ds((c0 + 1) * CH, CH)], eb1, sem1).wait()
        process(eb1)
        return 0

    lax.fori_loop(0, NCH // 2, pair, 0)
    pltpu.make_async_copy(packed_hbm.at[pl.ds(0, CH)], eb0, sem0).wait()

    base = w * PSLAB
    c1 = pltpu.async_copy(accv.at[pl.ds(0, PSLAB)], agg1_hbm.at[pl.ds(base, PSLAB)], sem0)
    c2 = pltpu.async_copy(accv.at[pl.ds(PSLAB, PSLAB)], agg2_hbm.at[pl.ds(base, PSLAB)], sem1)
    c1.wait()
    c2.wait()


def _sc_scatter(packed_flat, xflat):
    mesh = plsc.VectorSubcoreMesh(
        core_axis_name="c", subcore_axis_name="s",
        num_cores=NC, num_subcores=NS)
    f = functools.partial(
        pl.kernel,
        out_type=(
            jax.ShapeDtypeStruct((D_FEAT * N_PAD,), jnp.float32),
            jax.ShapeDtypeStruct((D_FEAT * N_PAD,), jnp.float32),
        ),
        mesh=mesh,
        compiler_params=pltpu.CompilerParams(needs_layout_passes=False),
        scratch_types=[
            pltpu.VMEM((SLAB,), jnp.float32),
            pltpu.VMEM((2 * PSLAB,), jnp.float32),
            pltpu.VMEM((CH,), jnp.int32),
            pltpu.VMEM((CH,), jnp.int32),
            pltpu.SemaphoreType.DMA,
            pltpu.SemaphoreType.DMA,
        ],
    )(_sc_body)
    return f(packed_flat, xflat)


def _head_body(a1_ref, a2_ref, w1t_ref, b1_ref, w2_ref, l1_ref, loss_ref):
    w1t = w1t_ref[...]
    b1 = b1_ref[...]
    w2 = w2_ref[...]
    valid = lax.broadcasted_iota(jnp.int32, (1, N_PAD), 1) < N_NODES
    h1 = jnp.maximum(
        jnp.dot(w1t, a1_ref[...], preferred_element_type=jnp.float32) + b1, 0.0)
    h2 = jnp.maximum(
        jnp.dot(w1t, a2_ref[...], preferred_element_type=jnp.float32) + b1, 0.0)
    s1 = jnp.sum(jnp.where(valid, h1 * w2, 0.0))
    s2 = jnp.sum(jnp.where(valid, h2 * w2, 0.0))
    pred1 = jax.nn.sigmoid(s1 / N_NODES)
    pred2 = jax.nn.sigmoid(s2 / N_NODES)
    bpr1 = jnp.maximum(GAM + 0.5 - pred1, 0.0)
    bpr2 = jnp.maximum(GAM + pred2 - 0.5, 0.0)
    loss_ref[...] = l1_ref[...] + LAM * (ALP * bpr1 + (1.0 - ALP) * bpr2)


def _head(a1, a2, w1t, b1c, w2c, l1):
    return pl.pallas_call(
        _head_body,
        out_shape=jax.ShapeDtypeStruct((1, 1), jnp.float32),
    )(a1, a2, w1t, b1c, w2c, l1)


def kernel(x, edge_mask, W1, b1, w2, edge_index):
    packed, ms, l1, expn = _pack(edge_index, edge_mask)

    xflat = x.T.reshape(-1)
    agg1f, agg2f = _sc_scatter(packed, xflat)

    loss = _head(agg1f.reshape(D_FEAT, N_PAD), agg2f.reshape(D_FEAT, N_PAD),
                 W1.T, b1.reshape(D_FEAT, 1), w2.reshape(D_FEAT, 1), l1)
    return loss[0, 0], ms, expn[0, 0]


# speedup vs baseline: 1.1007x; 1.0461x over previous
import functools

import jax
import jax.numpy as jnp
from jax import lax
from jax.experimental import pallas as pl
from jax.experimental.pallas import tpu as pltpu
from jax.experimental.pallas import tpu_sc as plsc

N_NODES = 10000
N_PAD = 10240
N_EDGES = 320000
D_FEAT = 128
GAM = 0.1
LAM = 1.0
ALP = 0.5

NC = 2
NS = 16
NW = NC * NS
FPW = D_FEAT // NW
SLAB = FPW * N_NODES
PSLAB = FPW * N_PAD
CH = 3200
NCH = N_EDGES // CH


def _pack_body(ei_ref, em_ref, x_ref, packed_ref, ms_ref, l1_ref, en_ref,
               xt_ref):
    em = em_ref[...]
    ms = jax.nn.sigmoid(em)
    keep = (em > 0.0).astype(jnp.int32)
    dstp = ei_ref[1, :] + keep * PSLAB
    packed_ref[...] = ei_ref[0, :] + dstp * 16384
    ms_ref[...] = ms
    l1_ref[...] = jnp.sum(ms)[None, None]
    en_ref[...] = jnp.sum(keep)[None, None]
    xt_ref[:, pl.ds(0, N_NODES)] = jnp.swapaxes(x_ref[...], 0, 1)


def _pack(edge_index, edge_mask, x):
    return pl.pallas_call(
        _pack_body,
        out_shape=(
            jax.ShapeDtypeStruct((N_EDGES,), jnp.int32),
            jax.ShapeDtypeStruct((N_EDGES,), jnp.float32),
            jax.ShapeDtypeStruct((1, 1), jnp.float32),
            jax.ShapeDtypeStruct((1, 1), jnp.int32),
            jax.ShapeDtypeStruct((D_FEAT, N_PAD), jnp.float32),
        ),
    )(edge_index, edge_mask, x)


def _sc_body(packed_hbm, xflat_hbm, agg1_hbm, agg2_hbm,
             xv, accv, eb0, eb1, sem0, sem1):
    w = lax.axis_index("s") * NC + lax.axis_index("c")

    pltpu.async_copy(packed_hbm.at[pl.ds(0, CH)], eb0, sem0)

    pltpu.sync_copy(xflat_hbm.at[pl.ds(w * FPW, FPW), :], xv)

    @plsc.parallel_loop(0, 2 * PSLAB, step=16, unroll=8)
    def _(i):
        accv[pl.ds(i, 16)] = jnp.zeros((16,), jnp.float32)

    def process(eb):
        @plsc.parallel_loop(0, CH, step=16, unroll=8)
        def _(i):
            p = eb[pl.ds(i, 16)]
            srci = lax.bitwise_and(p, 16383)
            dbase = lax.shift_right_logical(p, 14)
            for j in range(FPW):
                jv = jnp.full((16,), j, jnp.int32)
                v = plsc.load_gather(xv, [jv, srci])
                plsc.addupdate_scatter(accv, [dbase + (j * N_PAD)], v)

    def pair(cc, _):
        c0 = cc * 2
        pltpu.async_copy(packed_hbm.at[pl.ds((c0 + 1) * CH, CH)], eb1, sem1)
        pltpu.make_async_copy(packed_hbm.at[pl.ds(c0 * CH, CH)], eb0, sem0).wait()
        process(eb0)
        nxt = jnp.minimum(c0 + 2, NCH - 2)
        pltpu.async_copy(packed_hbm.at[pl.ds(nxt * CH, CH)], eb0, sem0)
        pltpu.make_async_copy(packed_hbm.at[pl.ds((c0 + 1) * CH, CH)], eb1, sem1).wait()
        process(eb1)
        return 0

    lax.fori_loop(0, NCH // 2, pair, 0)
    pltpu.make_async_copy(packed_hbm.at[pl.ds(0, CH)], eb0, sem0).wait()

    base = w * PSLAB
    c1 = pltpu.async_copy(accv.at[pl.ds(0, PSLAB)], agg1_hbm.at[pl.ds(base, PSLAB)], sem0)
    c2 = pltpu.async_copy(accv.at[pl.ds(PSLAB, PSLAB)], agg2_hbm.at[pl.ds(base, PSLAB)], sem1)
    c1.wait()
    c2.wait()


def _sc_scatter(packed_flat, xt):
    mesh = plsc.VectorSubcoreMesh(
        core_axis_name="c", subcore_axis_name="s",
        num_cores=NC, num_subcores=NS)
    f = functools.partial(
        pl.kernel,
        out_type=(
            jax.ShapeDtypeStruct((D_FEAT * N_PAD,), jnp.float32),
            jax.ShapeDtypeStruct((D_FEAT * N_PAD,), jnp.float32),
        ),
        mesh=mesh,
        compiler_params=pltpu.CompilerParams(needs_layout_passes=False),
        scratch_types=[
            pltpu.VMEM((FPW, N_PAD), jnp.float32),
            pltpu.VMEM((2 * PSLAB,), jnp.float32),
            pltpu.VMEM((CH,), jnp.int32),
            pltpu.VMEM((CH,), jnp.int32),
            pltpu.SemaphoreType.DMA,
            pltpu.SemaphoreType.DMA,
        ],
    )(_sc_body)
    return f(packed_flat, xt)


def _head_body(a1_ref, a2_ref, w1t_ref, b1_ref, w2_ref, l1_ref, loss_ref):
    w1t = w1t_ref[...]
    b1 = b1_ref[...]
    w2 = w2_ref[...]
    valid = lax.broadcasted_iota(jnp.int32, (1, N_PAD), 1) < N_NODES
    h1 = jnp.maximum(
        jnp.dot(w1t, a1_ref[...], preferred_element_type=jnp.float32) + b1, 0.0)
    h2 = jnp.maximum(
        jnp.dot(w1t, a2_ref[...], preferred_element_type=jnp.float32) + b1, 0.0)
    s1 = jnp.sum(jnp.where(valid, h1 * w2, 0.0))
    s2 = jnp.sum(jnp.where(valid, h2 * w2, 0.0))
    pred1 = jax.nn.sigmoid(s1 / N_NODES)
    pred2 = jax.nn.sigmoid(s2 / N_NODES)
    bpr1 = jnp.maximum(GAM + 0.5 - pred1, 0.0)
    bpr2 = jnp.maximum(GAM + pred2 - 0.5, 0.0)
    loss_ref[...] = l1_ref[...] + LAM * (ALP * bpr1 + (1.0 - ALP) * bpr2)


def _head(a1, a2, w1t, b1c, w2c, l1):
    return pl.pallas_call(
        _head_body,
        out_shape=jax.ShapeDtypeStruct((1, 1), jnp.float32),
    )(a1, a2, w1t, b1c, w2c, l1)


def kernel(x, edge_mask, W1, b1, w2, edge_index):
    packed, ms, l1, expn, xt = _pack(edge_index, edge_mask, x)

    agg1f, agg2f = _sc_scatter(packed, xt)

    loss = _head(agg1f.reshape(D_FEAT, N_PAD), agg2f.reshape(D_FEAT, N_PAD),
                 W1.T, b1.reshape(D_FEAT, 1), w2.reshape(D_FEAT, 1), l1)
    return loss[0, 0], ms, expn[0, 0]
